# baseline (speedup 1.0000x reference)
import functools
import numpy as np
import jax
import jax.numpy as jnp
from jax import lax
from jax.experimental import pallas as pl
from jax.experimental.pallas import tpu as pltpu
from jax.experimental.pallas import tpu_sc as plsc

_ALPHA = 4
_NBUF = 4
_HSPLIT = 2


def kernel(frames):
    B, T, C, H, W = frames.shape
    nsel = T // _ALPHA
    idx = [int(v) for v in np.linspace(0.0, T - 1, nsel).astype(np.int32)]
    unsel = [f for f in range(T) if f not in idx]
    Hc = H // _HSPLIT

    info = plsc.get_sparse_core_info()
    NW = info.num_cores * info.num_subcores
    n_pure = B * len(unsel) * C * _HSPLIT // NW
    n_fused = B * nsel * C * _HSPLIT // NW
    n_units = n_pure + n_fused

    def static_lookup(table, i):
        v = jnp.int32(0)
        for j, t in enumerate(table):
            v = v + jnp.where(i == j, t, 0)
        return v

    mesh = plsc.VectorSubcoreMesh(core_axis_name="c", subcore_axis_name="s")

    @functools.partial(
        pl.kernel,
        mesh=mesh,
        out_type=[
            jax.ShapeDtypeStruct((B, nsel, C, H, W), frames.dtype),
            jax.ShapeDtypeStruct((B, T, C, H, W), frames.dtype),
        ],
        scratch_types=(
            [pltpu.VMEM((Hc, W), frames.dtype) for _ in range(_NBUF)]
            + [pltpu.SemaphoreType.DMA for _ in range(3 * _NBUF)]
        ),
    )
    def pack_k(frames_hbm, slow_hbm, fast_hbm, *scratch):
        bufs = scratch[:_NBUF]
        in_sems = scratch[_NBUF:2 * _NBUF]
        fast_sems = scratch[2 * _NBUF:3 * _NBUF]
        slow_sems = scratch[3 * _NBUF:4 * _NBUF]
        wid = lax.axis_index("s") * info.num_cores + lax.axis_index("c")

        def unit(i):
            if i < n_pure:
                u = wid * n_pure + i
                h = (u % _HSPLIT) * Hc
                c = (u // _HSPLIT) % C
                fpos = (u // (_HSPLIT * C)) % len(unsel)
                b = u // (_HSPLIT * C * len(unsel))
                f = static_lookup(unsel, fpos)
                return (frames_hbm.at[b, f, c, pl.ds(h, Hc)],
                        fast_hbm.at[b, f, c, pl.ds(h, Hc)], None)
            u = wid * n_fused + (i - n_pure)
            h = (u % _HSPLIT) * Hc
            c = (u // _HSPLIT) % C
            s = (u // (_HSPLIT * C)) % nsel
            b = u // (_HSPLIT * C * nsel)
            f = static_lookup(idx, s)
            return (frames_hbm.at[b, f, c, pl.ds(h, Hc)],
                    fast_hbm.at[b, f, c, pl.ds(h, Hc)],
                    slow_hbm.at[b, s, c, pl.ds(h, Hc)])

        in_flight = [None] * n_units
        out_flight = [None] * n_units

        src0, _, _ = unit(0)
        in_flight[0] = pltpu.async_copy(src0, bufs[0], in_sems[0])
        for i in range(n_units):
            bi = i % _NBUF
            if i - (_NBUF - 1) >= 0:
                for cp in out_flight[i - (_NBUF - 1)]:
                    cp.wait()
            if i + 1 < n_units:
                src, _, _ = unit(i + 1)
                nbi = (i + 1) % _NBUF
                in_flight[i + 1] = pltpu.async_copy(src, bufs[nbi], in_sems[nbi])
            in_flight[i].wait()
            _, fast_dst, slow_dst = unit(i)
            outs = [pltpu.async_copy(bufs[bi], fast_dst, fast_sems[bi])]
            if slow_dst is not None:
                outs.append(pltpu.async_copy(bufs[bi], slow_dst, slow_sems[bi]))
            out_flight[i] = outs
        for j in range(max(0, n_units - (_NBUF - 1)), n_units):
            for cp in out_flight[j]:
                cp.wait()

    slow, fast = pack_k(frames)
    return (slow, fast)


# speedup vs baseline: 1.3612x; 1.3612x over previous
import numpy as np
import jax
import jax.numpy as jnp
from jax.experimental import pallas as pl

_ALPHA = 4


def kernel(frames):
    B, T, C, H, W = frames.shape
    nsel = T // _ALPHA
    idx = [int(v) for v in np.linspace(0.0, T - 1, nsel).astype(np.int32)]
    TB = 2
    ngrp = T // TB

    def slot_of(g):
        s = 0
        for v in idx:
            s = s + jnp.where(g * TB > v, 1, 0)
        return s

    pos_in_grp = [0] * ngrp
    for v in idx:
        pos_in_grp[v // TB] = v % TB

    def body(x_ref, slow_ref, fast_ref):
        g = pl.program_id(0)
        v = x_ref[...]
        fast_ref[...] = v
        pos = 0
        for gi in range(ngrp):
            pos = pos + jnp.where(g == gi, pos_in_grp[gi], 0)
        slow_ref[...] = jnp.where(pos == 0, v[:, 0:1], v[:, 1:2])

    slow, fast = pl.pallas_call(
        body,
        grid=(ngrp,),
        in_specs=[pl.BlockSpec((B, TB, C, H, W), lambda g: (0, g, 0, 0, 0))],
        out_specs=[
            pl.BlockSpec((B, 1, C, H, W), lambda g: (0, slot_of(g), 0, 0, 0)),
            pl.BlockSpec((B, TB, C, H, W), lambda g: (0, g, 0, 0, 0)),
        ],
        out_shape=[
            jax.ShapeDtypeStruct((B, nsel, C, H, W), frames.dtype),
            jax.ShapeDtypeStruct((B, T, C, H, W), frames.dtype),
        ],
    )(frames)
    return (slow, fast)
